# baseline (speedup 1.0000x reference)
import functools

import jax
import jax.numpy as jnp
from jax import lax
from jax.experimental import pallas as pl
from jax.experimental.pallas import tpu as pltpu
from jax.experimental.pallas import tpu_sc as plsc

N = 10000
E = 320000
F_IN = 128
HID = 128
C_OUT = 40
C_PAD = 48

NC = 2
NS = 16
NW = NC * NS
K = 128
NCHT = 84
EPT = NCHT * K
E_PAD = NW * EPT
EXTRA = E_PAD - E
NBUF = 2

NPAD = 10240
RPT = NPAD // NS
ZROWS = 64
ZCOPIES = RPT // ZROWS

_mesh = plsc.VectorSubcoreMesh(core_axis_name="c", subcore_axis_name="s")


def _worker_id():
    return lax.axis_index("s") * NC + lax.axis_index("c")


def _fill_zeros_2d(ref, rows, cols):
    def body(t, _):
        r = t // (cols // 16)
        c = t % (cols // 16)
        ref[r, pl.ds(c * 16, 16)] = jnp.zeros((16,), jnp.float32)
        return 0
    lax.fori_loop(0, rows * (cols // 16), body, 0)


def _fill_1d(ref, n, value):
    def body(t, _):
        ref[pl.ds(t * 16, 16)] = jnp.full((16,), value, jnp.float32)
        return 0
    lax.fori_loop(0, n // 16, body, 0)


def _gather_scatter_pipeline(g_hbm, row_hbm, col_hbm, wid, rows_v, col_v,
                             bufs, acc, sem_g, ncht):
    pltpu.sync_copy(row_hbm.at[wid, 0], rows_v[0])
    pltpu.async_copy(g_hbm.at[rows_v[0]], bufs[0], sem_g[0])
    pltpu.sync_copy(row_hbm.at[wid, 1], rows_v[1])
    pltpu.async_copy(g_hbm.at[rows_v[1]], bufs[1], sem_g[1])

    def body(j, _):
        for b in range(NBUF):
            c = NBUF * j + b
            pltpu.make_async_copy(g_hbm.at[rows_v[b]], bufs[b],
                                  sem_g[b]).wait()
            pltpu.sync_copy(col_hbm.at[wid, c], col_v)
            pltpu.sync_copy(bufs[b], acc.at[col_v], add=True)

            @pl.when(c + NBUF < ncht)
            def _():
                pltpu.sync_copy(row_hbm.at[wid, c + NBUF], rows_v[b])
                pltpu.async_copy(g_hbm.at[rows_v[b]], bufs[b], sem_g[b])
        return 0
    lax.fori_loop(0, ncht // NBUF, body, 0)


@functools.partial(
    pl.kernel,
    out_type=jax.ShapeDtypeStruct((NC, NPAD), jnp.float32),
    mesh=_mesh,
    scratch_types=[
        pltpu.VMEM((K,), jnp.int32),
        pltpu.VMEM((K,), jnp.float32),
        pltpu.VMEM((RPT,), jnp.float32),
        pltpu.VMEM_SHARED((NPAD,), jnp.float32),
    ],
)
def _sc_deg(col_hbm, out_hbm, col_v, ones_v, zero_v, acc):
    cid = lax.axis_index("c")
    sid = lax.axis_index("s")
    wid = _worker_id()

    _fill_1d(ones_v, K, 1.0)
    _fill_1d(zero_v, RPT, 0.0)
    pltpu.sync_copy(zero_v, acc.at[pl.ds(sid * RPT, RPT)])
    plsc.subcore_barrier()

    def body(i, _):
        pltpu.sync_copy(col_hbm.at[wid, i], col_v)
        pltpu.sync_copy(ones_v, acc.at[col_v], add=True)
        return 0
    lax.fori_loop(0, NCHT, body, 0)

    plsc.subcore_barrier()
    pltpu.sync_copy(acc.at[pl.ds(sid * RPT, RPT)], zero_v)
    pltpu.sync_copy(zero_v, out_hbm.at[cid, pl.ds(sid * RPT, RPT)])


def _make_prop(d):
  @functools.partial(
    pl.kernel,
    out_type=jax.ShapeDtypeStruct((NC, NPAD, d), jnp.float32),
    mesh=_mesh,
    scratch_types=[
        [pltpu.VMEM((K,), jnp.int32)] * NBUF,
        pltpu.VMEM((K,), jnp.int32),
        [pltpu.VMEM((K, d), jnp.float32)] * NBUF,
        pltpu.VMEM((ZROWS, d), jnp.float32),
        pltpu.VMEM_SHARED((NPAD, d), jnp.float32),
        [pltpu.SemaphoreType.DMA] * NBUF,
    ],
    compiler_params=pltpu.CompilerParams(use_tc_tiling_on_sc=False),
  )
  def prop(g_hbm, row_hbm, col_hbm, out_hbm,
           rows_v, col_v, bufs, zero_v, acc, sem_g):
      cid = lax.axis_index("c")
      sid = lax.axis_index("s")
      wid = _worker_id()

      _fill_zeros_2d(zero_v, ZROWS, d)

      def zbody(j, _):
          pltpu.sync_copy(zero_v, acc.at[pl.ds(sid * RPT + j * ZROWS, ZROWS)])
          return 0
      lax.fori_loop(0, ZCOPIES, zbody, 0)
      plsc.subcore_barrier()

      _gather_scatter_pipeline(g_hbm, row_hbm, col_hbm, wid, rows_v, col_v,
                               bufs, acc, sem_g, NCHT)
      plsc.subcore_barrier()

      def wbody(j, _):
          base = sid * RPT + j * ZROWS
          pltpu.sync_copy(acc.at[pl.ds(base, ZROWS)], zero_v)
          pltpu.sync_copy(zero_v, out_hbm.at[cid, pl.ds(base, ZROWS)])
          return 0
      lax.fori_loop(0, ZCOPIES, wbody, 0)
  return prop


_sc_prop1 = _make_prop(HID)
_sc_prop2 = _make_prop(C_PAD)


BLK = 1000
GRID = N // BLK


def _tc1_body(deg_ref, x_ref, w_ref, g_ref, dinv_ref):
    deg = deg_ref[0] + deg_ref[1] + 1.0
    dinv = lax.rsqrt(deg)
    h = jnp.dot(x_ref[...], w_ref[...], preferred_element_type=jnp.float32)
    g_ref[...] = h * dinv
    dinv_ref[...] = dinv


def _tc1(deg2, x, w1t):
    return pl.pallas_call(
        _tc1_body,
        grid=(GRID,),
        in_specs=[
            pl.BlockSpec((NC, BLK, 1), lambda i: (0, i, 0)),
            pl.BlockSpec((BLK, F_IN), lambda i: (i, 0)),
            pl.BlockSpec((F_IN, HID), lambda i: (0, 0)),
        ],
        out_specs=[
            pl.BlockSpec((BLK, HID), lambda i: (i, 0)),
            pl.BlockSpec((BLK, 1), lambda i: (i, 0)),
        ],
        out_shape=[
            jax.ShapeDtypeStruct((N, HID), jnp.float32),
            jax.ShapeDtypeStruct((N, 1), jnp.float32),
        ],
    )(deg2, x, w1t)


def _tc2_body(p_ref, g1_ref, dinv_ref, b1_ref, w_ref, g2_ref):
    s = p_ref[0] + p_ref[1] + g1_ref[...]
    out1 = jnp.maximum(dinv_ref[...] * s + b1_ref[...], 0.0)
    h2 = jnp.dot(out1, w_ref[...], preferred_element_type=jnp.float32)
    g2_ref[...] = h2 * dinv_ref[...]


def _tc2(p1, g1, dinv, b1, w2t):
    return pl.pallas_call(
        _tc2_body,
        grid=(GRID,),
        in_specs=[
            pl.BlockSpec((NC, BLK, HID), lambda i: (0, i, 0)),
            pl.BlockSpec((BLK, HID), lambda i: (i, 0)),
            pl.BlockSpec((BLK, 1), lambda i: (i, 0)),
            pl.BlockSpec((1, HID), lambda i: (0, 0)),
            pl.BlockSpec((HID, C_PAD), lambda i: (0, 0)),
        ],
        out_specs=pl.BlockSpec((BLK, C_PAD), lambda i: (i, 0)),
        out_shape=jax.ShapeDtypeStruct((N, C_PAD), jnp.float32),
    )(p1, g1, dinv, b1, w2t)


def _tc3_body(q_ref, g2_ref, dinv_ref, b2_ref, out_ref):
    s = q_ref[0] + q_ref[1] + g2_ref[...]
    out_ref[...] = dinv_ref[...] * s + b2_ref[...]


def _tc3(q, g2, dinv, b2):
    return pl.pallas_call(
        _tc3_body,
        grid=(GRID,),
        in_specs=[
            pl.BlockSpec((NC, BLK, C_PAD), lambda i: (0, i, 0)),
            pl.BlockSpec((BLK, C_PAD), lambda i: (i, 0)),
            pl.BlockSpec((BLK, 1), lambda i: (i, 0)),
            pl.BlockSpec((1, C_PAD), lambda i: (0, 0)),
        ],
        out_specs=pl.BlockSpec((BLK, C_PAD), lambda i: (i, 0)),
        out_shape=jax.ShapeDtypeStruct((N, C_PAD), jnp.float32),
    )(q, g2, dinv, b2)


def kernel(x, edge_index, W1, b1, W2, b2):
    row = jnp.concatenate(
        [edge_index[0], jnp.zeros((EXTRA,), jnp.int32)]).reshape(NW, NCHT, K)
    col = jnp.concatenate(
        [edge_index[1], jnp.full((EXTRA,), NPAD - 1, jnp.int32)]
    ).reshape(NW, NCHT, K)

    deg2 = _sc_deg(col).reshape(NC, NPAD, 1)
    g1, dinv = _tc1(deg2, x, W1.T)
    p1 = _sc_prop1(g1, row, col)

    w2t = jnp.zeros((HID, C_PAD), jnp.float32).at[:, :C_OUT].set(W2.T)
    b1r = b1.reshape(1, HID)
    b2r = jnp.zeros((1, C_PAD), jnp.float32).at[0, :C_OUT].set(b2)

    g2 = _tc2(p1, g1, dinv, b1r, w2t)
    q = _sc_prop2(g2, row, col)
    out = _tc3(q, g2, dinv, b2r)
    return out[:, :C_OUT]


# speedup vs baseline: 2.1762x; 2.1762x over previous
import functools

import jax
import jax.numpy as jnp
from jax import lax
from jax.experimental import pallas as pl
from jax.experimental.pallas import tpu as pltpu
from jax.experimental.pallas import tpu_sc as plsc

N = 10000
E = 320000
F_IN = 128
HID = 128
C_OUT = 40
C_PAD = 48

NC = 2
NS = 16
NW = NC * NS
EPT = E // NW
K = 80
NCH = EPT // K

NPAD = 10240
RPT = NPAD // NS
ZROWS = 64
ZCOPIES = RPT // ZROWS

_mesh = plsc.VectorSubcoreMesh(core_axis_name="c", subcore_axis_name="s")


def _worker_id():
    return lax.axis_index("s") * NC + lax.axis_index("c")


def _fill_zeros_2d(ref, rows, cols):
    def body(t, _):
        r = t // (cols // 16)
        c = t % (cols // 16)
        ref[r, pl.ds(c * 16, 16)] = jnp.zeros((16,), jnp.float32)
        return 0
    lax.fori_loop(0, rows * (cols // 16), body, 0)


def _fill_1d(ref, n, value):
    def body(t, _):
        ref[pl.ds(t * 16, 16)] = jnp.full((16,), value, jnp.float32)
        return 0
    lax.fori_loop(0, n // 16, body, 0)


@functools.partial(
    pl.kernel,
    out_type=jax.ShapeDtypeStruct((NC, NPAD), jnp.float32),
    mesh=_mesh,
    scratch_types=[
        pltpu.VMEM((K,), jnp.int32),
        pltpu.VMEM((K,), jnp.float32),
        pltpu.VMEM((RPT,), jnp.float32),
        pltpu.VMEM_SHARED((NPAD,), jnp.float32),
    ],
)
def _sc_deg(col_hbm, out_hbm, col_v, ones_v, zero_v, acc):
    cid = lax.axis_index("c")
    sid = lax.axis_index("s")
    wid = _worker_id()

    _fill_1d(ones_v, K, 1.0)
    _fill_1d(zero_v, RPT, 0.0)
    pltpu.sync_copy(zero_v, acc.at[pl.ds(sid * RPT, RPT)])
    plsc.subcore_barrier()

    def body(i, _):
        pltpu.sync_copy(col_hbm.at[wid, i], col_v)
        pltpu.sync_copy(ones_v, acc.at[col_v], add=True)
        return 0
    lax.fori_loop(0, NCH, body, 0)

    plsc.subcore_barrier()
    pltpu.sync_copy(acc.at[pl.ds(sid * RPT, RPT)],
                    out_hbm.at[cid, pl.ds(sid * RPT, RPT)])


def _make_prop(d):
    @functools.partial(
        pl.kernel,
        out_type=jax.ShapeDtypeStruct((NC, NPAD, d), jnp.float32),
        mesh=_mesh,
        scratch_types=[
            pltpu.VMEM((K,), jnp.int32),
            pltpu.VMEM((K,), jnp.int32),
            pltpu.VMEM((K, d), jnp.float32),
            pltpu.VMEM((ZROWS, d), jnp.float32),
            pltpu.VMEM_SHARED((NPAD, d), jnp.float32),
            pltpu.SemaphoreType.DMA,
        ],
        compiler_params=pltpu.CompilerParams(use_tc_tiling_on_sc=False),
    )
    def prop(g_hbm, row_hbm, col_hbm, out_hbm,
             row_v, col_v, rows_v, zero_v, acc, sem):
        cid = lax.axis_index("c")
        sid = lax.axis_index("s")
        wid = _worker_id()

        _fill_zeros_2d(zero_v, ZROWS, d)

        def zbody(j, _):
            pltpu.sync_copy(zero_v, acc.at[pl.ds(sid * RPT + j * ZROWS, ZROWS)])
            return 0
        lax.fori_loop(0, ZCOPIES, zbody, 0)
        plsc.subcore_barrier()

        def body(i, _):
            pltpu.sync_copy(row_hbm.at[wid, i], row_v)
            pltpu.sync_copy(col_hbm.at[wid, i], col_v)
            pltpu.async_copy(g_hbm.at[row_v], rows_v, sem).wait()
            pltpu.sync_copy(rows_v, acc.at[col_v], add=True)
            return 0
        lax.fori_loop(0, NCH, body, 0)

        plsc.subcore_barrier()
        pltpu.sync_copy(acc.at[pl.ds(sid * RPT, RPT)],
                        out_hbm.at[cid, pl.ds(sid * RPT, RPT)])
    return prop


_sc_prop128 = _make_prop(HID)
_sc_prop48 = _make_prop(C_PAD)


BLK = 1000
GRID = N // BLK


def _tc1_body(deg_ref, x_ref, w_ref, g_ref, dinv_ref):
    deg = deg_ref[0] + deg_ref[1] + 1.0
    dinv = lax.rsqrt(deg)
    h = jnp.dot(x_ref[...], w_ref[...], preferred_element_type=jnp.float32)
    g_ref[...] = h * dinv
    dinv_ref[...] = dinv


def _tc1(deg2, x, w1t):
    return pl.pallas_call(
        _tc1_body,
        grid=(GRID,),
        in_specs=[
            pl.BlockSpec((NC, BLK, 1), lambda i: (0, i, 0)),
            pl.BlockSpec((BLK, F_IN), lambda i: (i, 0)),
            pl.BlockSpec((F_IN, HID), lambda i: (0, 0)),
        ],
        out_specs=[
            pl.BlockSpec((BLK, HID), lambda i: (i, 0)),
            pl.BlockSpec((BLK, 1), lambda i: (i, 0)),
        ],
        out_shape=[
            jax.ShapeDtypeStruct((N, HID), jnp.float32),
            jax.ShapeDtypeStruct((N, 1), jnp.float32),
        ],
    )(deg2, x, w1t)


def _tc2_body(p_ref, g1_ref, dinv_ref, b1_ref, w_ref, g2_ref):
    s = p_ref[0] + p_ref[1] + g1_ref[...]
    out1 = jnp.maximum(dinv_ref[...] * s + b1_ref[...], 0.0)
    h2 = jnp.dot(out1, w_ref[...], preferred_element_type=jnp.float32)
    g2_ref[...] = h2 * dinv_ref[...]


def _tc2(p1, g1, dinv, b1, w2t):
    return pl.pallas_call(
        _tc2_body,
        grid=(GRID,),
        in_specs=[
            pl.BlockSpec((NC, BLK, HID), lambda i: (0, i, 0)),
            pl.BlockSpec((BLK, HID), lambda i: (i, 0)),
            pl.BlockSpec((BLK, 1), lambda i: (i, 0)),
            pl.BlockSpec((1, HID), lambda i: (0, 0)),
            pl.BlockSpec((HID, C_PAD), lambda i: (0, 0)),
        ],
        out_specs=pl.BlockSpec((BLK, C_PAD), lambda i: (i, 0)),
        out_shape=jax.ShapeDtypeStruct((N, C_PAD), jnp.float32),
    )(p1, g1, dinv, b1, w2t)


def _tc3_body(q_ref, g2_ref, dinv_ref, b2_ref, out_ref):
    s = q_ref[0] + q_ref[1] + g2_ref[...]
    out_ref[...] = dinv_ref[...] * s + b2_ref[...]


def _tc3(q, g2, dinv, b2):
    return pl.pallas_call(
        _tc3_body,
        grid=(GRID,),
        in_specs=[
            pl.BlockSpec((NC, BLK, C_PAD), lambda i: (0, i, 0)),
            pl.BlockSpec((BLK, C_PAD), lambda i: (i, 0)),
            pl.BlockSpec((BLK, 1), lambda i: (i, 0)),
            pl.BlockSpec((1, C_PAD), lambda i: (0, 0)),
        ],
        out_specs=pl.BlockSpec((BLK, C_PAD), lambda i: (i, 0)),
        out_shape=jax.ShapeDtypeStruct((N, C_PAD), jnp.float32),
    )(q, g2, dinv, b2)


def kernel(x, edge_index, W1, b1, W2, b2):
    row = edge_index[0].reshape(NW, NCH, K)
    col = edge_index[1].reshape(NW, NCH, K)

    deg2 = _sc_deg(col).reshape(NC, NPAD, 1)
    g1, dinv = _tc1(deg2, x, W1.T)
    p1 = _sc_prop128(g1, row, col)

    w2t = jnp.zeros((HID, C_PAD), jnp.float32).at[:, :C_OUT].set(W2.T)
    b1r = b1.reshape(1, HID)
    b2r = jnp.zeros((1, C_PAD), jnp.float32).at[0, :C_OUT].set(b2)

    g2 = _tc2(p1, g1, dinv, b1r, w2t)
    q = _sc_prop48(g2, row, col)
    out = _tc3(q, g2, dinv, b2r)
    return out[:, :C_OUT]
